# baseline (speedup 1.0000x reference)
import jax
import jax.numpy as jnp
from jax import lax
from jax.experimental import pallas as pl
from jax.experimental.pallas import tpu as pltpu
from jax.experimental.pallas import tpu_sc as plsc

NTOK = 1000
B, L = 1024, 50
NC, NS = 2, 16
BPW = B // (NC * NS)
SB = 2
NSLAB = BPW // SB
TOK = SB * L
LANES = 16
LA = 48
VA = 896
GROUPS = (0, 16, 32, 48, 64, 80, TOK - LANES)


def _zero_rows(ref, rows, cols):
    z = jnp.zeros((LANES,), jnp.float32)

    def _row(r):
        for c in range(cols // LANES):
            ref[r // rows, r % rows, pl.ds(c * LANES, LANES)] = z
        if cols % LANES:
            ref[r // rows, r % rows, pl.ds(cols - LANES, LANES)] = z

    pl.loop(0, SB * rows)(_row)


def _body(x_hbm, out_hbm, xv, bufm, bufa, bufb, sem):
    wid = lax.axis_index("c") * NS + lax.axis_index("s")
    b0 = wid * BPW

    pltpu.sync_copy(x_hbm.at[pl.ds(b0 * L, BPW * L)], xv)

    _zero_rows(bufm, LA, VA)
    _zero_rows(bufa, LA, NTOK - VA)
    _zero_rows(bufb, L - LA, NTOK)

    lane = lax.iota(jnp.int32, LANES)
    ones = jnp.ones((LANES,), jnp.float32)
    zeros = jnp.zeros((LANES,), jnp.float32)

    def scatter(i, value):
        for l0 in GROUPS:
            j = l0 + lane
            bb = j // L
            l = j % L
            v = xv[pl.ds(i * TOK + l0, LANES)]
            ok = (v >= 0) & (v < NTOK)
            inm = ok & (l < LA) & (v < VA)
            ina = ok & (l < LA) & (v >= VA)
            inb = ok & (l >= LA)
            plsc.store_scatter(
                bufm, [bb, jnp.where(inm, l, 0), jnp.where(inm, v, 0)],
                value, mask=inm)
            plsc.store_scatter(
                bufa, [bb, jnp.where(ina, l, 0), jnp.where(ina, v - VA, 0)],
                value, mask=ina)
            plsc.store_scatter(
                bufb, [bb, jnp.where(inb, l - LA, 0), jnp.where(inb, v, 0)],
                value, mask=inb)

    for i in range(NSLAB):
        bb0 = b0 + i * SB
        scatter(i, ones)
        pltpu.sync_copy(bufm, out_hbm.at[pl.ds(bb0, SB), pl.ds(0, LA),
                                         pl.ds(0, VA)])
        pltpu.sync_copy(bufa, out_hbm.at[pl.ds(bb0, SB), pl.ds(0, LA),
                                         pl.ds(VA, NTOK - VA)])
        pltpu.sync_copy(bufb, out_hbm.at[pl.ds(bb0, SB), pl.ds(LA, L - LA),
                                         pl.ds(0, NTOK)])
        scatter(i, zeros)


@jax.jit
def kernel(x):
    mesh = plsc.VectorSubcoreMesh(
        core_axis_name="c", subcore_axis_name="s",
        num_cores=NC, num_subcores=NS,
    )
    run = pl.kernel(
        _body,
        out_type=jax.ShapeDtypeStruct((B, L, NTOK), jnp.float32),
        mesh=mesh,
        scratch_types=[
            pltpu.VMEM((BPW * L,), jnp.int32),
            pltpu.VMEM((SB, LA, VA), jnp.float32),
            pltpu.VMEM((SB, LA, NTOK - VA), jnp.float32),
            pltpu.VMEM((SB, L - LA, NTOK), jnp.float32),
            pltpu.SemaphoreType.DMA,
        ],
        compiler_params=pltpu.CompilerParams(
            needs_layout_passes=False,
            use_tc_tiling_on_sc=True,
        ),
    )
    return run(x.reshape(B * L).astype(jnp.int32))
